# initial kernel scaffold (unmeasured)
import functools

import jax
import jax.numpy as jnp
from jax import lax
from jax.experimental import pallas as pl
from jax.experimental.pallas import tpu as pltpu

N_DEV = 8
M = 4096
N = 8192
CHUNK = M // N_DEV
N_TILES = 4
TILE_N = N // N_TILES


def _body(x_ref, w_ref, out_ref, send_buf, recv_buf, send_sems, recv_sems,
          store_sem, credit_sem):
    d = lax.axis_index("i")
    right = lax.rem(d + 1, N_DEV)
    left = lax.rem(d + N_DEV - 1, N_DEV)

    barrier_sem = pltpu.get_barrier_semaphore()
    for nbr in (left, right):
        pl.semaphore_signal(barrier_sem, inc=1, device_id=(nbr,),
                            device_id_type=pl.DeviceIdType.MESH)
    pl.semaphore_wait(barrier_sem, 2)

    def matmul_chunk(c, add_from_slot):
        xc = x_ref[pl.ds(c * CHUNK, CHUNK), :]
        for j in range(N_TILES):
            acc = jnp.dot(xc, w_ref[:, j * TILE_N:(j + 1) * TILE_N],
                          preferred_element_type=jnp.float32)
            if add_from_slot is not None:
                acc = acc + recv_buf[add_from_slot, :,
                                     j * TILE_N:(j + 1) * TILE_N].astype(
                                         jnp.float32)
            send_buf[:, j * TILE_N:(j + 1) * TILE_N] = acc.astype(jnp.bfloat16)

    def ring_send(step):
        slot = step % 2
        if step >= 2:
            pl.semaphore_wait(credit_sem, 1)
        rdma = pltpu.make_async_remote_copy(
            src_ref=send_buf,
            dst_ref=recv_buf.at[slot],
            send_sem=send_sems.at[slot],
            recv_sem=recv_sems.at[slot],
            device_id=(right,),
            device_id_type=pl.DeviceIdType.MESH,
        )
        rdma.start()
        rdma.wait()
        return slot

    def give_credit():
        pl.semaphore_signal(credit_sem, inc=1, device_id=(left,),
                            device_id_type=pl.DeviceIdType.MESH)

    matmul_chunk(d, None)
    for s in range(N_DEV - 1):
        slot = ring_send(s)
        c = lax.rem(d + (2 * N_DEV - 1 - s), N_DEV)
        matmul_chunk(c, slot)
        give_credit()

    g0 = lax.rem(d + 1, N_DEV)
    cp = pltpu.make_async_copy(send_buf, out_ref.at[pl.ds(g0 * CHUNK, CHUNK), :],
                               store_sem)
    cp.start()
    cp.wait()

    for t in range(N_DEV - 1):
        slot = ring_send(N_DEV - 1 + t)
        g = lax.rem(d + (2 * N_DEV - t), N_DEV)
        cp = pltpu.make_async_copy(recv_buf.at[slot],
                                   out_ref.at[pl.ds(g * CHUNK, CHUNK), :],
                                   store_sem)
        cp.start()
        cp.wait()
        if t < N_DEV - 2:
            send_buf[:, :] = recv_buf[slot]
        give_credit()

    pl.semaphore_wait(credit_sem, 2)


def kernel(x, w_mat):
    y = pl.pallas_call(
        _body,
        out_shape=jax.ShapeDtypeStruct((M, N), jnp.bfloat16),
        in_specs=[
            pl.BlockSpec(memory_space=pltpu.VMEM),
            pl.BlockSpec(memory_space=pltpu.VMEM),
        ],
        out_specs=pl.BlockSpec(memory_space=pltpu.ANY),
        scratch_shapes=[
            pltpu.VMEM((CHUNK, N), jnp.bfloat16),
            pltpu.VMEM((2, CHUNK, N), jnp.bfloat16),
            pltpu.SemaphoreType.DMA((2,)),
            pltpu.SemaphoreType.DMA((2,)),
            pltpu.SemaphoreType.DMA,
            pltpu.SemaphoreType.REGULAR,
        ],
        compiler_params=pltpu.CompilerParams(collective_id=0),
    )(x, w_mat)

    y32 = y.astype(jnp.float32)
    scale = jnp.max(jnp.abs(y32)) / 448.0
    q = (y32 / scale).astype(jnp.float8_e4m3fn)
    return q.astype(jnp.float32) * scale


# baseline (device time: 1517000 ns/iter reference)
import functools

import jax
import jax.numpy as jnp
from jax import lax
from jax.experimental import pallas as pl
from jax.experimental.pallas import tpu as pltpu

N_DEV = 8
M = 4096
N = 8192
CHUNK = M // N_DEV
N_TILES = 4
TILE_N = N // N_TILES


def _body(x_ref, w_ref, out_ref, send_buf, recv_buf, send_sems, recv_sems,
          store_sem, credit_sem):
    d = lax.axis_index("i")
    right = lax.rem(d + 1, N_DEV)
    left = lax.rem(d + N_DEV - 1, N_DEV)

    barrier_sem = pltpu.get_barrier_semaphore()
    for nbr in (left, right):
        pl.semaphore_signal(barrier_sem, inc=1, device_id=(nbr,),
                            device_id_type=pl.DeviceIdType.MESH)
    pl.semaphore_wait(barrier_sem, 2)

    def matmul_chunk(c, add_from_slot):
        xc = x_ref[pl.ds(c * CHUNK, CHUNK), :]
        for j in range(N_TILES):
            acc = jnp.dot(xc, w_ref[:, j * TILE_N:(j + 1) * TILE_N],
                          preferred_element_type=jnp.float32)
            if add_from_slot is not None:
                acc = acc + recv_buf[add_from_slot, :,
                                     j * TILE_N:(j + 1) * TILE_N].astype(
                                         jnp.float32)
            send_buf[:, j * TILE_N:(j + 1) * TILE_N] = acc.astype(jnp.bfloat16)

    def ring_send(step):
        slot = step % 2
        if step >= 2:
            pl.semaphore_wait(credit_sem, 1)
        rdma = pltpu.make_async_remote_copy(
            src_ref=send_buf,
            dst_ref=recv_buf.at[slot],
            send_sem=send_sems.at[slot],
            recv_sem=recv_sems.at[slot],
            device_id=(right,),
            device_id_type=pl.DeviceIdType.MESH,
        )
        rdma.start()
        rdma.wait()
        return slot

    def give_credit():
        pl.semaphore_signal(credit_sem, inc=1, device_id=(left,),
                            device_id_type=pl.DeviceIdType.MESH)

    matmul_chunk(d, None)
    for s in range(N_DEV - 1):
        slot = ring_send(s)
        c = lax.rem(d + (2 * N_DEV - 1 - s), N_DEV)
        matmul_chunk(c, slot)
        give_credit()

    g0 = lax.rem(d + 1, N_DEV)
    cp = pltpu.make_async_copy(send_buf, out_ref.at[pl.ds(g0 * CHUNK, CHUNK), :],
                               store_sem)
    cp.start()
    cp.wait()

    for t in range(N_DEV - 1):
        slot = ring_send(N_DEV - 1 + t)
        g = lax.rem(d + (2 * N_DEV - t), N_DEV)
        cp = pltpu.make_async_copy(recv_buf.at[slot],
                                   out_ref.at[pl.ds(g * CHUNK, CHUNK), :],
                                   store_sem)
        cp.start()
        cp.wait()
        if t < N_DEV - 2:
            send_buf[:, :] = recv_buf[slot]
        give_credit()

    pl.semaphore_wait(credit_sem, 2)


def kernel(x, w_mat):
    x = x.astype(jnp.bfloat16)
    w_mat = w_mat.astype(jnp.bfloat16)
    y = pl.pallas_call(
        _body,
        out_shape=jax.ShapeDtypeStruct((M, N), jnp.bfloat16),
        in_specs=[
            pl.BlockSpec(memory_space=pltpu.VMEM),
            pl.BlockSpec(memory_space=pltpu.VMEM),
        ],
        out_specs=pl.BlockSpec(memory_space=pl.ANY),
        scratch_shapes=[
            pltpu.VMEM((CHUNK, N), jnp.bfloat16),
            pltpu.VMEM((2, CHUNK, N), jnp.bfloat16),
            pltpu.SemaphoreType.DMA((2,)),
            pltpu.SemaphoreType.DMA((2,)),
            pltpu.SemaphoreType.DMA,
            pltpu.SemaphoreType.REGULAR,
        ],
        compiler_params=pltpu.CompilerParams(
            collective_id=0, vmem_limit_bytes=100 * 1024 * 1024),
    )(x, w_mat)

    y32 = y.astype(jnp.float32)
    scale = jnp.max(jnp.abs(y32)) / 448.0
    z = y32 / scale
    a = jnp.abs(z)
    e = (jax.lax.bitcast_convert_type(a, jnp.int32) >> 23) - 127
    step = jnp.where(a >= 2.0 ** -6,
                     jnp.exp2((e - 3).astype(jnp.float32)),
                     jnp.float32(2.0 ** -9))
    snapped = jnp.minimum(jnp.round(a / step) * step, 448.0)
    return jnp.sign(z) * snapped * scale


# device time: 843142 ns/iter; 1.7992x vs baseline; 1.7992x over previous
import jax
import jax.numpy as jnp
from jax import lax
from jax.experimental import pallas as pl
from jax.experimental.pallas import tpu as pltpu

N_DEV = 8
M = 4096
N = 8192
HALF = N // 2
CHUNK = M // N_DEV
TILE_N = 2048
N_TILES = HALF // TILE_N


def _body(x_ref, w_ref, out_ref,
          send_r, send_l, recv_r, recv_l, acc_r, acc_l,
          send_sems_r, recv_sems_r, send_sems_l, recv_sems_l,
          store_sems, credit_r, credit_l):
    d = lax.axis_index("i")
    right = lax.rem(d + 1, N_DEV)
    left = lax.rem(d + N_DEV - 1, N_DEV)

    barrier_sem = pltpu.get_barrier_semaphore()
    for nbr in (left, right):
        pl.semaphore_signal(barrier_sem, inc=1, device_id=(nbr,),
                            device_id_type=pl.DeviceIdType.MESH)
    pl.semaphore_wait(barrier_sem, 2)

    def partial_into(acc, c, col0):
        xc = x_ref[pl.ds(c * CHUNK, CHUNK), :]
        for j in range(N_TILES):
            acc[:, j * TILE_N:(j + 1) * TILE_N] = jnp.dot(
                xc, w_ref[:, col0 + j * TILE_N:col0 + (j + 1) * TILE_N],
                preferred_element_type=jnp.float32)

    def rdma(src, dst_slot_r, dst_slot_l, slot, t0=False):
        r = pltpu.make_async_remote_copy(
            src_ref=send_r if t0 else src[0],
            dst_ref=recv_r.at[dst_slot_r],
            send_sem=send_sems_r.at[slot], recv_sem=recv_sems_r.at[slot],
            device_id=(right,), device_id_type=pl.DeviceIdType.MESH)
        l = pltpu.make_async_remote_copy(
            src_ref=send_l if t0 else src[1],
            dst_ref=recv_l.at[dst_slot_l],
            send_sem=send_sems_l.at[slot], recv_sem=recv_sems_l.at[slot],
            device_id=(left,), device_id_type=pl.DeviceIdType.MESH)
        return r, l

    def give_credits():
        pl.semaphore_signal(credit_r, inc=1, device_id=(left,),
                            device_id_type=pl.DeviceIdType.MESH)
        pl.semaphore_signal(credit_l, inc=1, device_id=(right,),
                            device_id_type=pl.DeviceIdType.MESH)

    def wait_credits():
        pl.semaphore_wait(credit_r, 1)
        pl.semaphore_wait(credit_l, 1)

    partial_into(acc_r, d, 0)
    partial_into(acc_l, d, HALF)
    send_r[:, :] = acc_r[:, :].astype(jnp.bfloat16)
    send_l[:, :] = acc_l[:, :].astype(jnp.bfloat16)

    for s in range(N_DEV - 1):
        slot = s % 2
        if s >= 2:
            wait_credits()
        rr, rl = rdma((send_r, send_l), slot, slot, slot)
        rr.start()
        rl.start()
        cr = lax.rem(d + (2 * N_DEV - 1 - s), N_DEV)
        cl = lax.rem(d + s + 1, N_DEV)
        partial_into(acc_r, cr, 0)
        partial_into(acc_l, cl, HALF)
        rr.wait()
        rl.wait()
        send_r[:, :] = (acc_r[:, :]
                        + recv_r[slot].astype(jnp.float32)).astype(jnp.bfloat16)
        send_l[:, :] = (acc_l[:, :]
                        + recv_l[slot].astype(jnp.float32)).astype(jnp.bfloat16)
        give_credits()

    g_r = lax.rem(d + 1, N_DEV)
    g_l = lax.rem(d + N_DEV - 1, N_DEV)
    st_r = pltpu.make_async_copy(
        send_r, out_ref.at[pl.ds(g_r * CHUNK, CHUNK), pl.ds(0, HALF)],
        store_sems.at[0])
    st_l = pltpu.make_async_copy(
        send_l, out_ref.at[pl.ds(g_l * CHUNK, CHUNK), pl.ds(HALF, HALF)],
        store_sems.at[1])
    st_r.start()
    st_l.start()

    prev_st = (st_r, st_l)
    for t in range(N_DEV - 1):
        slot = (t + 1) % 2
        wait_credits()
        rr, rl = rdma((recv_r.at[1 - slot], recv_l.at[1 - slot]),
                      slot, slot, slot, t0=(t == 0))
        rr.start()
        rl.start()
        rr.wait()
        rl.wait()
        g_r = lax.rem(d + (2 * N_DEV - t), N_DEV)
        g_l = lax.rem(d + t, N_DEV)
        st_r = pltpu.make_async_copy(
            recv_r.at[slot],
            out_ref.at[pl.ds(g_r * CHUNK, CHUNK), pl.ds(0, HALF)],
            store_sems.at[0])
        st_l = pltpu.make_async_copy(
            recv_l.at[slot],
            out_ref.at[pl.ds(g_l * CHUNK, CHUNK), pl.ds(HALF, HALF)],
            store_sems.at[1])
        prev_st[0].wait()
        prev_st[1].wait()
        st_r.start()
        st_l.start()
        prev_st = (st_r, st_l)
        if t >= 1:
            give_credits()

    prev_st[0].wait()
    prev_st[1].wait()
    give_credits()
    pl.semaphore_wait(credit_r, 2)
    pl.semaphore_wait(credit_l, 2)


def kernel(x, w_mat):
    x = x.astype(jnp.bfloat16)
    w_mat = w_mat.astype(jnp.bfloat16)
    y = pl.pallas_call(
        _body,
        out_shape=jax.ShapeDtypeStruct((M, N), jnp.bfloat16),
        in_specs=[
            pl.BlockSpec(memory_space=pltpu.VMEM),
            pl.BlockSpec(memory_space=pltpu.VMEM),
        ],
        out_specs=pl.BlockSpec(memory_space=pl.ANY),
        scratch_shapes=[
            pltpu.VMEM((CHUNK, HALF), jnp.bfloat16),
            pltpu.VMEM((CHUNK, HALF), jnp.bfloat16),
            pltpu.VMEM((2, CHUNK, HALF), jnp.bfloat16),
            pltpu.VMEM((2, CHUNK, HALF), jnp.bfloat16),
            pltpu.VMEM((CHUNK, HALF), jnp.float32),
            pltpu.VMEM((CHUNK, HALF), jnp.float32),
            pltpu.SemaphoreType.DMA((2,)),
            pltpu.SemaphoreType.DMA((2,)),
            pltpu.SemaphoreType.DMA((2,)),
            pltpu.SemaphoreType.DMA((2,)),
            pltpu.SemaphoreType.DMA((2,)),
            pltpu.SemaphoreType.REGULAR,
            pltpu.SemaphoreType.REGULAR,
        ],
        compiler_params=pltpu.CompilerParams(
            collective_id=0, vmem_limit_bytes=100 * 1024 * 1024),
    )(x, w_mat)

    y32 = y.astype(jnp.float32)
    scale = jnp.max(jnp.abs(y32)) / 448.0
    z = y32 / scale
    a = jnp.abs(z)
    e = (jax.lax.bitcast_convert_type(a, jnp.int32) >> 23) - 127
    step = jnp.where(a >= 2.0 ** -6,
                     jnp.exp2((e - 3).astype(jnp.float32)),
                     jnp.float32(2.0 ** -9))
    snapped = jnp.minimum(jnp.round(a / step) * step, 448.0)
    return jnp.sign(z) * snapped * scale


# device time: 821603 ns/iter; 1.8464x vs baseline; 1.0262x over previous
import jax
import jax.numpy as jnp
from jax import lax
from jax.experimental import pallas as pl
from jax.experimental.pallas import tpu as pltpu

N_DEV = 8
M = 4096
N = 8192
HALF = N // 2
CHUNK = M // N_DEV
TILE_N = 2048
N_TILES = HALF // TILE_N


def _body(x_ref, w_ref, out_ref,
          send_r, send_l, recv_r, recv_l, acc_r, acc_l,
          amax_buf,
          send_sems_r, recv_sems_r, send_sems_l, recv_sems_l,
          store_sems_r, store_sems_l, amax_send_sems, amax_recv_sems,
          credit_r, credit_l):
    d = lax.axis_index("i")
    right = lax.rem(d + 1, N_DEV)
    left = lax.rem(d + N_DEV - 1, N_DEV)

    barrier_sem = pltpu.get_barrier_semaphore()
    for nbr in (left, right):
        pl.semaphore_signal(barrier_sem, inc=1, device_id=(nbr,),
                            device_id_type=pl.DeviceIdType.MESH)
    pl.semaphore_wait(barrier_sem, 2)

    def partial_into(acc, c, col0):
        xc = x_ref[pl.ds(c * CHUNK, CHUNK), :]
        for j in range(N_TILES):
            acc[:, j * TILE_N:(j + 1) * TILE_N] = jnp.dot(
                xc, w_ref[:, col0 + j * TILE_N:col0 + (j + 1) * TILE_N],
                preferred_element_type=jnp.float32)

    def rdma_pair(src_r, src_l, slot):
        r = pltpu.make_async_remote_copy(
            src_ref=src_r, dst_ref=recv_r.at[slot],
            send_sem=send_sems_r.at[slot], recv_sem=recv_sems_r.at[slot],
            device_id=(right,), device_id_type=pl.DeviceIdType.MESH)
        l = pltpu.make_async_remote_copy(
            src_ref=src_l, dst_ref=recv_l.at[slot],
            send_sem=send_sems_l.at[slot], recv_sem=recv_sems_l.at[slot],
            device_id=(left,), device_id_type=pl.DeviceIdType.MESH)
        return r, l

    def give_credits():
        pl.semaphore_signal(credit_r, inc=1, device_id=(left,),
                            device_id_type=pl.DeviceIdType.MESH)
        pl.semaphore_signal(credit_l, inc=1, device_id=(right,),
                            device_id_type=pl.DeviceIdType.MESH)

    def wait_credits():
        pl.semaphore_wait(credit_r, 1)
        pl.semaphore_wait(credit_l, 1)

    partial_into(acc_r, d, 0)
    partial_into(acc_l, d, HALF)
    send_r[:, :] = acc_r[:, :].astype(jnp.bfloat16)
    send_l[:, :] = acc_l[:, :].astype(jnp.bfloat16)

    for s in range(N_DEV - 1):
        slot = s % 2
        if s >= 2:
            wait_credits()
        rr, rl = rdma_pair(send_r, send_l, slot)
        rr.start()
        rl.start()
        cr = lax.rem(d + (2 * N_DEV - 1 - s), N_DEV)
        cl = lax.rem(d + s + 1, N_DEV)
        partial_into(acc_r, cr, 0)
        partial_into(acc_l, cl, HALF)
        rr.wait()
        rl.wait()
        send_r[:, :] = (acc_r[:, :]
                        + recv_r[slot].astype(jnp.float32)).astype(jnp.bfloat16)
        send_l[:, :] = (acc_l[:, :]
                        + recv_l[slot].astype(jnp.float32)).astype(jnp.bfloat16)
        give_credits()


    my_max = jnp.maximum(jnp.max(jnp.abs(send_r[:, :].astype(jnp.float32))),
                         jnp.max(jnp.abs(send_l[:, :].astype(jnp.float32))))
    amax_buf[N_DEV - 1] = jnp.full((8, 128), my_max, jnp.float32)
    for h in range(N_DEV - 1):
        am = pltpu.make_async_remote_copy(
            src_ref=amax_buf.at[N_DEV - 1],
            dst_ref=amax_buf.at[h],
            send_sem=amax_send_sems.at[h],
            recv_sem=amax_recv_sems.at[h],
            device_id=(right,), device_id_type=pl.DeviceIdType.MESH)
        am.start()
        am.wait()
        amax_buf[N_DEV - 1] = jnp.maximum(amax_buf[N_DEV - 1],
                                          amax_buf[h])
    scale = jnp.max(amax_buf[N_DEV - 1]) / 448.0

    def dequant_into(stg, chunk_bf16):
        for j in range(N_TILES):
            js = slice(j * TILE_N, (j + 1) * TILE_N)
            z = chunk_bf16[:, js].astype(jnp.float32) / scale
            a = jnp.abs(z)
            u = jax.lax.bitcast_convert_type(a, jnp.int32)
            r = (u + 0x7FFFF + ((u >> 20) & 1)) & ~0xFFFFF
            an = jax.lax.bitcast_convert_type(r, jnp.float32)
            asub = jnp.round(a * 512.0) * (1.0 / 512.0)
            snapped = jnp.minimum(jnp.where(a >= 2.0 ** -6, an, asub), 448.0)
            stg[:, js] = jnp.where(z < 0, -snapped, snapped) * scale

    def store(stg, g, col0, sem):
        cp = pltpu.make_async_copy(
            stg, out_ref.at[pl.ds(g * CHUNK, CHUNK), pl.ds(col0, HALF)], sem)
        cp.start()
        return cp

    pend = [None, None]

    def process(chunk_r, chunk_l, g_r, g_l, ev):
        if pend[0] is not None:
            pend[0].wait()
        if pend[1] is not None:
            pend[1].wait()
        dequant_into(acc_r, chunk_r)
        dequant_into(acc_l, chunk_l)
        pend[0] = store(acc_r, g_r, 0, store_sems_r.at[ev % 2])
        pend[1] = store(acc_l, g_l, HALF, store_sems_l.at[ev % 2])

    for t in range(N_DEV - 1):
        slot = (t + 1) % 2
        wait_credits()
        if t == 0:
            rr, rl = rdma_pair(send_r, send_l, slot)
        else:
            rr, rl = rdma_pair(recv_r.at[1 - slot], recv_l.at[1 - slot], slot)
        rr.start()
        rl.start()
        if t == 0:
            process(send_r, send_l,
                    lax.rem(d + 1, N_DEV), lax.rem(d + N_DEV - 1, N_DEV), 0)
        else:
            process(recv_r.at[1 - slot], recv_l.at[1 - slot],
                    lax.rem(d + (2 * N_DEV - t) + 1, N_DEV),
                    lax.rem(d + t - 1, N_DEV), t)
        rr.wait()
        rl.wait()
        if t >= 1:
            give_credits()

    last_slot = (N_DEV - 1) % 2
    process(recv_r.at[last_slot], recv_l.at[last_slot],
            lax.rem(d + (2 * N_DEV - 6), N_DEV), lax.rem(d + 6, N_DEV), 7)
    for cp in pend:
        cp.wait()
    give_credits()
    pl.semaphore_wait(credit_r, 2)
    pl.semaphore_wait(credit_l, 2)


def kernel(x, w_mat):
    x = x.astype(jnp.bfloat16)
    w_mat = w_mat.astype(jnp.bfloat16)
    return pl.pallas_call(
        _body,
        out_shape=jax.ShapeDtypeStruct((M, N), jnp.float32),
        in_specs=[
            pl.BlockSpec(memory_space=pltpu.VMEM),
            pl.BlockSpec(memory_space=pltpu.VMEM),
        ],
        out_specs=pl.BlockSpec(memory_space=pl.ANY),
        scratch_shapes=[
            pltpu.VMEM((CHUNK, HALF), jnp.bfloat16),
            pltpu.VMEM((CHUNK, HALF), jnp.bfloat16),
            pltpu.VMEM((2, CHUNK, HALF), jnp.bfloat16),
            pltpu.VMEM((2, CHUNK, HALF), jnp.bfloat16),
            pltpu.VMEM((CHUNK, HALF), jnp.float32),
            pltpu.VMEM((CHUNK, HALF), jnp.float32),
            pltpu.VMEM((N_DEV, 8, 128), jnp.float32),
            pltpu.SemaphoreType.DMA((2,)),
            pltpu.SemaphoreType.DMA((2,)),
            pltpu.SemaphoreType.DMA((2,)),
            pltpu.SemaphoreType.DMA((2,)),
            pltpu.SemaphoreType.DMA((2,)),
            pltpu.SemaphoreType.DMA((2,)),
            pltpu.SemaphoreType.DMA((N_DEV - 1,)),
            pltpu.SemaphoreType.DMA((N_DEV - 1,)),
            pltpu.SemaphoreType.REGULAR,
            pltpu.SemaphoreType.REGULAR,
        ],
        compiler_params=pltpu.CompilerParams(
            collective_id=0, vmem_limit_bytes=128 * 1024 * 1024),
    )(x, w_mat)
